# baseline (speedup 1.0000x reference)
import functools
import math

import jax
import jax.numpy as jnp
from jax import lax
from jax.experimental import pallas as pl
from jax.experimental.pallas import tpu as pltpu
from jax.experimental.pallas import tpu_sc as plsc

_LS = 0.1
_CONF = 1.0 - _LS
_EPS = 1e-12
_V = 100000
_B = 1024
_G = _V - 100
_GOFF = (_G // 128) * 128
_GPOS = _G - _GOFF
_S = _LS / (_V - 1)
_LOG_S = math.log(_S)
_LOG_C = math.log(_CONF)
_R = 64
_NB = _B // _R

_NW = 32
_RPW = _B // _NW
_L = 16


@functools.partial(
    pl.kernel,
    mesh=plsc.VectorSubcoreMesh(core_axis_name="c", subcore_axis_name="s"),
    out_type=jax.ShapeDtypeStruct((_B, 8, 128), jnp.float32),
    scratch_types=[
        pltpu.VMEM((_RPW,), jnp.int32),
        pltpu.SemaphoreType.DMA,
    ],
)
def _sc_gather(x_hbm, t_hbm, comp_hbm, t_v, sem):
    wid = lax.axis_index("s") * 2 + lax.axis_index("c")
    base = pl.multiple_of(wid * _RPW, _RPW)
    pltpu.sync_copy(t_hbm.at[pl.ds(base, _RPW)], t_v)

    copies = []
    for k in range(_RPW // _L):
        tk = t_v[pl.ds(k * _L, _L)]
        for m in range(_L):
            j = k * _L + m
            t = jnp.maximum(tk[m], 0)
            toff = pl.multiple_of((t >> 7) << 7, 128)
            roff = pl.multiple_of(base + (j // 8) * 8, 8)
            copies.append(pltpu.async_copy(
                x_hbm.at[pl.ds(roff, 8), pl.ds(toff, 128)],
                comp_hbm.at[base + j], sem))
    for c in copies:
        c.wait()


def _rowsum_kernel(x_ref, s_ref):
    l = jnp.log(x_ref[...] + _EPS)
    s_ref[...] = jnp.sum(l, axis=1, keepdims=True)


def _combine_kernel(s_ref, c_ref, xg_ref, t_ref, out_ref):
    s = s_ref[...]
    t = t_ref[...]
    comp = c_ref[...]
    t3 = jnp.reshape(jnp.maximum(t, 0), (_B, 1, 1))
    row_in_tile = jax.lax.broadcasted_iota(jnp.int32, (_B, 8, 128), 0) & 7
    sub = jax.lax.broadcasted_iota(jnp.int32, (_B, 8, 128), 1)
    lane = jax.lax.broadcasted_iota(jnp.int32, (_B, 8, 128), 2)
    pick = (sub == row_in_tile) & (lane == (t3 & 127))
    xt = jnp.sum(jnp.sum(jnp.where(pick, comp, 0.0), axis=2), axis=1,
                 keepdims=True)
    lt = jnp.log(xt + _EPS)
    lg = jnp.log(xg_ref[:, _GPOS:_GPOS + 1] + _EPS)
    valid = (t != -100).astype(jnp.float32)
    delta = (t != _G).astype(jnp.float32)
    const = _S * _LOG_S * (_V - 1 - delta) + _CONF * _LOG_C
    row = -_S * s + (_S - _CONF) * lt + _S * delta * lg + const
    out_ref[...] = jnp.reshape(jnp.sum(valid * row), (1, 1))


def kernel(output, target):
    comp = _sc_gather(output, target)

    s_rows = pl.pallas_call(
        _rowsum_kernel,
        grid=(_NB,),
        in_specs=[pl.BlockSpec((_R, _V), lambda i: (i, 0))],
        out_specs=pl.BlockSpec((_R, 1), lambda i: (i, 0)),
        out_shape=jax.ShapeDtypeStruct((_B, 1), jnp.float32),
    )(output)

    out = pl.pallas_call(
        _combine_kernel,
        grid=(1,),
        in_specs=[
            pl.BlockSpec((_B, 1), lambda i: (0, 0)),
            pl.BlockSpec((_B, 8, 128), lambda i: (0, 0, 0)),
            pl.BlockSpec((_B, 128), lambda i: (0, _GOFF // 128)),
            pl.BlockSpec((_B, 1), lambda i: (0, 0)),
        ],
        out_specs=pl.BlockSpec((1, 1), lambda i: (0, 0)),
        out_shape=jax.ShapeDtypeStruct((1, 1), jnp.float32),
    )(s_rows, comp, output, target.reshape(_B, 1))
    return out[0, 0]


# speedup vs baseline: 1.0040x; 1.0040x over previous
import functools
import math

import jax
import jax.numpy as jnp
from jax import lax
from jax.experimental import pallas as pl
from jax.experimental.pallas import tpu as pltpu
from jax.experimental.pallas import tpu_sc as plsc

_LS = 0.1
_CONF = 1.0 - _LS
_EPS = 1e-12
_V = 100000
_B = 1024
_G = _V - 100
_GOFF = (_G // 128) * 128
_GPOS = _G - _GOFF
_S = _LS / (_V - 1)
_LOG_S = math.log(_S)
_LOG_C = math.log(_CONF)
_R = 64
_NB = _B // _R

_NW = 32
_RPW = _B // _NW
_L = 16


@functools.partial(
    pl.kernel,
    mesh=plsc.VectorSubcoreMesh(core_axis_name="c", subcore_axis_name="s"),
    out_type=jax.ShapeDtypeStruct((_B, 128), jnp.float32),
    scratch_types=[
        pltpu.VMEM((_RPW,), jnp.int32),
        pltpu.VMEM((_RPW, 8, 128), jnp.float32),
        pltpu.VMEM((_RPW, 128), jnp.float32),
        pltpu.SemaphoreType.DMA,
    ],
)
def _sc_gather(x_hbm, t_hbm, comp_hbm, t_v, win_t, asm, sem):
    wid = lax.axis_index("s") * 2 + lax.axis_index("c")
    base = pl.multiple_of(wid * _RPW, _RPW)
    pltpu.sync_copy(t_hbm.at[pl.ds(base, _RPW)], t_v)

    copies = []
    for k in range(_RPW // _L):
        tk = t_v[pl.ds(k * _L, _L)]
        for m in range(_L):
            j = k * _L + m
            t = jnp.maximum(tk[m], 0)
            toff = pl.multiple_of((t >> 7) << 7, 128)
            roff = pl.multiple_of(base + (j // 8) * 8, 8)
            copies.append(pltpu.async_copy(
                x_hbm.at[pl.ds(roff, 8), pl.ds(toff, 128)],
                win_t.at[j], sem))
    for c in copies:
        c.wait()
    for j in range(_RPW):
        for c in range(128 // _L):
            asm[j, pl.ds(c * _L, _L)] = win_t[j, j % 8, pl.ds(c * _L, _L)]
    pltpu.sync_copy(asm, comp_hbm.at[pl.ds(base, _RPW)])


def _rowsum_kernel(x_ref, s_ref):
    l = jnp.log(x_ref[...] + _EPS)
    s_ref[...] = jnp.sum(l, axis=1, keepdims=True)


def _combine_kernel(s_ref, c_ref, xg_ref, t_ref, out_ref):
    s = s_ref[...]
    t = t_ref[...]
    comp = c_ref[...]
    lane = jax.lax.broadcasted_iota(jnp.int32, (_B, 128), 1)
    pick = lane == (jnp.maximum(t, 0) & 127)
    xt = jnp.sum(jnp.where(pick, comp, 0.0), axis=1,
                 keepdims=True)
    lt = jnp.log(xt + _EPS)
    lg = jnp.log(xg_ref[:, _GPOS:_GPOS + 1] + _EPS)
    valid = (t != -100).astype(jnp.float32)
    delta = (t != _G).astype(jnp.float32)
    const = _S * _LOG_S * (_V - 1 - delta) + _CONF * _LOG_C
    row = -_S * s + (_S - _CONF) * lt + _S * delta * lg + const
    out_ref[...] = jnp.reshape(jnp.sum(valid * row), (1, 1))


def kernel(output, target):
    comp = _sc_gather(output, target)

    s_rows = pl.pallas_call(
        _rowsum_kernel,
        grid=(_NB,),
        in_specs=[pl.BlockSpec((_R, _V), lambda i: (i, 0))],
        out_specs=pl.BlockSpec((_R, 1), lambda i: (i, 0)),
        out_shape=jax.ShapeDtypeStruct((_B, 1), jnp.float32),
    )(output)

    out = pl.pallas_call(
        _combine_kernel,
        grid=(1,),
        in_specs=[
            pl.BlockSpec((_B, 1), lambda i: (0, 0)),
            pl.BlockSpec((_B, 128), lambda i: (0, 0)),
            pl.BlockSpec((_B, 128), lambda i: (0, _GOFF // 128)),
            pl.BlockSpec((_B, 1), lambda i: (0, 0)),
        ],
        out_specs=pl.BlockSpec((1, 1), lambda i: (0, 0)),
        out_shape=jax.ShapeDtypeStruct((1, 1), jnp.float32),
    )(s_rows, comp, output, target.reshape(_B, 1))
    return out[0, 0]


# speedup vs baseline: 1.0085x; 1.0045x over previous
import functools
import math

import jax
import jax.numpy as jnp
from jax import lax
from jax.experimental import pallas as pl
from jax.experimental.pallas import tpu as pltpu
from jax.experimental.pallas import tpu_sc as plsc

_LS = 0.1
_CONF = 1.0 - _LS
_EPS = 1e-12
_V = 100000
_B = 1024
_G = _V - 100
_GOFF = (_G // 128) * 128
_GPOS = _G - _GOFF
_S = _LS / (_V - 1)
_LOG_S = math.log(_S)
_LOG_C = math.log(_CONF)
_R = 64
_NB = _B // _R

_NW = 32
_RPW = _B // _NW
_L = 16


@functools.partial(
    pl.kernel,
    mesh=plsc.VectorSubcoreMesh(core_axis_name="c", subcore_axis_name="s"),
    out_type=jax.ShapeDtypeStruct((_B, 128), jnp.float32),
    scratch_types=[
        pltpu.VMEM((_RPW,), jnp.int32),
        pltpu.VMEM((_RPW, 8, 128), jnp.float32),
        pltpu.VMEM((_RPW, 128), jnp.float32),
        pltpu.SemaphoreType.DMA,
    ],
)
def _sc_gather(x_hbm, t_hbm, comp_hbm, t_v, win_t, asm, sem):
    wid = lax.axis_index("s") * 2 + lax.axis_index("c")
    base = pl.multiple_of(wid * _RPW, _RPW)
    pltpu.sync_copy(t_hbm.at[pl.ds(base, _RPW)], t_v)

    copies = []
    for k in range(_RPW // _L):
        tk = t_v[pl.ds(k * _L, _L)]
        for m in range(_L):
            j = k * _L + m
            t = jnp.maximum(tk[m], 0)
            toff = pl.multiple_of((t >> 7) << 7, 128)
            roff = pl.multiple_of(base + (j // 8) * 8, 8)
            copies.append(pltpu.async_copy(
                x_hbm.at[pl.ds(roff, 8), pl.ds(toff, 128)],
                win_t.at[j], sem))
    for c in copies:
        c.wait()
    for j in range(_RPW):
        for c in range(128 // _L):
            asm[j, pl.ds(c * _L, _L)] = win_t[j, j % 8, pl.ds(c * _L, _L)]
    pltpu.sync_copy(asm, comp_hbm.at[pl.ds(base, _RPW)])


def _rowsum_kernel(x_ref, s_ref, lg_ref):
    x = x_ref[...]
    l = jnp.log(x + _EPS)
    s_ref[...] = jnp.sum(l, axis=1, keepdims=True)
    lg_ref[...] = jnp.log(x[:, _G:_G + 1] + _EPS)


def _combine_kernel(s_ref, c_ref, lg_ref, t_ref, out_ref):
    s = s_ref[...]
    t = t_ref[...]
    comp = c_ref[...]
    lane = jax.lax.broadcasted_iota(jnp.int32, (_B, 128), 1)
    pick = lane == (jnp.maximum(t, 0) & 127)
    xt = jnp.sum(jnp.where(pick, comp, 0.0), axis=1,
                 keepdims=True)
    lt = jnp.log(xt + _EPS)
    lg = lg_ref[...]
    valid = (t != -100).astype(jnp.float32)
    delta = (t != _G).astype(jnp.float32)
    const = _S * _LOG_S * (_V - 1 - delta) + _CONF * _LOG_C
    row = -_S * s + (_S - _CONF) * lt + _S * delta * lg + const
    out_ref[...] = jnp.reshape(jnp.sum(valid * row), (1, 1))


def kernel(output, target):
    comp = _sc_gather(output, target)

    s_rows, lg_rows = pl.pallas_call(
        _rowsum_kernel,
        grid=(_NB,),
        in_specs=[pl.BlockSpec((_R, _V), lambda i: (i, 0))],
        out_specs=[pl.BlockSpec((_R, 1), lambda i: (i, 0)),
                   pl.BlockSpec((_R, 1), lambda i: (i, 0))],
        out_shape=[jax.ShapeDtypeStruct((_B, 1), jnp.float32),
                   jax.ShapeDtypeStruct((_B, 1), jnp.float32)],
    )(output)

    out = pl.pallas_call(
        _combine_kernel,
        grid=(1,),
        in_specs=[
            pl.BlockSpec((_B, 1), lambda i: (0, 0)),
            pl.BlockSpec((_B, 128), lambda i: (0, 0)),
            pl.BlockSpec((_B, 1), lambda i: (0, 0)),
            pl.BlockSpec((_B, 1), lambda i: (0, 0)),
        ],
        out_specs=pl.BlockSpec((1, 1), lambda i: (0, 0)),
        out_shape=jax.ShapeDtypeStruct((1, 1), jnp.float32),
    )(s_rows, comp, lg_rows, target.reshape(_B, 1))
    return out[0, 0]
